# baseline (speedup 1.0000x reference)
import functools
import jax
import jax.numpy as jnp
from jax.experimental import pallas as pl
from jax.experimental.pallas import tpu as pltpu


def _matmul3_kernel(nf_ref, ws_ref, wr_ref, wn_ref, sp_ref, rp_ref, np_ref):
    x = nf_ref[...]
    sp_ref[...] = jax.lax.dot_general(x, ws_ref[...], (((1,), (1,)), ((), ())),
                                      preferred_element_type=jnp.float32)
    rp_ref[...] = jax.lax.dot_general(x, wr_ref[...], (((1,), (1,)), ((), ())),
                                      preferred_element_type=jnp.float32)
    np_ref[...] = jax.lax.dot_general(x, wn_ref[...], (((1,), (1,)), ((), ())),
                                      preferred_element_type=jnp.float32)


def _node_projections(nf, f_sender, f_receiver, g_node):
    n, d = nf.shape
    blk = 1024
    grid = (pl.cdiv(n, blk),)
    out = pl.pallas_call(
        _matmul3_kernel,
        grid=grid,
        in_specs=[
            pl.BlockSpec((blk, d), lambda i: (i, 0)),
            pl.BlockSpec((128, d), lambda i: (0, 0)),
            pl.BlockSpec((128, d), lambda i: (0, 0)),
            pl.BlockSpec((128, d), lambda i: (0, 0)),
        ],
        out_specs=[
            pl.BlockSpec((blk, 128), lambda i: (i, 0)),
            pl.BlockSpec((blk, 128), lambda i: (i, 0)),
            pl.BlockSpec((blk, 128), lambda i: (i, 0)),
        ],
        out_shape=[jax.ShapeDtypeStruct((n, 128), jnp.float32)] * 3,
    )(nf, f_sender, f_receiver, g_node)
    return out


def _edge_proj_kernel(ef_ref, we_ref, b_ref, out_ref):
    out_ref[...] = jax.lax.dot_general(
        ef_ref[...], we_ref[...], (((1,), (1,)), ((), ())),
        preferred_element_type=jnp.float32) + b_ref[...]


def _edge_projection(ef, f_edge, f_bias):
    e, d = ef.shape
    blk = 2048
    out = pl.pallas_call(
        _edge_proj_kernel,
        grid=(pl.cdiv(e, blk),),
        in_specs=[
            pl.BlockSpec((blk, d), lambda i: (i, 0)),
            pl.BlockSpec((128, d), lambda i: (0, 0)),
            pl.BlockSpec((1, 128), lambda i: (0, 0)),
        ],
        out_specs=pl.BlockSpec((blk, 128), lambda i: (i, 0)),
        out_shape=jax.ShapeDtypeStruct((e, 128), jnp.float32),
    )(ef, f_edge, f_bias.reshape(1, 128))
    return out


def _final_kernel(nf_ref, inc_ref, out_ref, gids_ref, gn_ref, gi_ref, go_ref,
                  gb_ref, nodes_ref, nagg_ref):
    i = pl.program_id(0)
    dot = functools.partial(jax.lax.dot_general, preferred_element_type=jnp.float32)
    nodes = (dot(nf_ref[...], gn_ref[...], (((1,), (1,)), ((), ())))
             + dot(inc_ref[...], gi_ref[...], (((1,), (1,)), ((), ())))
             + dot(out_ref[...], go_ref[...], (((1,), (1,)), ((), ())))
             + gb_ref[...])
    nodes_ref[...] = nodes
    onehot = (gids_ref[...] == jax.lax.broadcasted_iota(jnp.int32, (16, 1), 0)
              ).astype(jnp.float32)
    part = dot(onehot, jnp.maximum(nodes, 0.0), (((1,), (0,)), ((), ())))

    @pl.when(i == 0)
    def _init():
        nagg_ref[...] = jnp.zeros_like(nagg_ref)

    nagg_ref[...] += part


def _final_dense(nf, incoming, outgoing, node_gids, g_node, g_in, g_out, g_bias):
    n = nf.shape[0]
    blk = 1024
    grid = (pl.cdiv(n, blk),)
    npad = grid[0] * blk
    pad = npad - n
    nf = jnp.pad(nf, ((0, pad), (0, 0)))
    incoming = jnp.pad(incoming, ((0, pad), (0, 0)))
    outgoing = jnp.pad(outgoing, ((0, pad), (0, 0)))
    gids = jnp.full((1, npad), 16, jnp.int32).at[0, :n].set(node_gids.astype(jnp.int32))
    nodes, nagg = pl.pallas_call(
        _final_kernel,
        grid=grid,
        in_specs=[
            pl.BlockSpec((blk, 128), lambda i: (i, 0)),
            pl.BlockSpec((blk, 128), lambda i: (i, 0)),
            pl.BlockSpec((blk, 128), lambda i: (i, 0)),
            pl.BlockSpec((1, blk), lambda i: (0, i)),
            pl.BlockSpec((128, 128), lambda i: (0, 0)),
            pl.BlockSpec((128, 128), lambda i: (0, 0)),
            pl.BlockSpec((128, 128), lambda i: (0, 0)),
            pl.BlockSpec((1, 128), lambda i: (0, 0)),
        ],
        out_specs=[
            pl.BlockSpec((blk, 128), lambda i: (i, 0)),
            pl.BlockSpec((16, 128), lambda i: (0, 0)),
        ],
        out_shape=[
            jax.ShapeDtypeStruct((npad, 128), jnp.float32),
            jax.ShapeDtypeStruct((16, 128), jnp.float32),
        ],
    )(nf, incoming, outgoing, gids, g_node, g_in, g_out, g_bias.reshape(1, 128))
    return nodes[:n], nagg


def _globals_kernel(eagg_ref, nagg_ref, he_ref, hn_ref, hb_ref, out_ref):
    dot = functools.partial(jax.lax.dot_general, preferred_element_type=jnp.float32)
    out_ref[...] = (dot(eagg_ref[...], he_ref[...], (((1,), (1,)), ((), ())))
                    + dot(nagg_ref[...], hn_ref[...], (((1,), (1,)), ((), ())))
                    + hb_ref[...])


def _globals_dense(eagg, nagg, h_edges, h_nodes, h_bias):
    return pl.pallas_call(
        _globals_kernel,
        out_shape=jax.ShapeDtypeStruct((16, 128), jnp.float32),
    )(eagg, nagg, h_edges, h_nodes, h_bias.reshape(1, 128))


def kernel(node_features, edge_features, senders, receivers, node_graph_ids,
           edge_graph_ids, f_edge, f_sender, f_receiver, f_bias, g_node, g_in,
           g_out, g_bias, h_nodes, h_edges, h_bias):
    n = node_features.shape[0]
    sp, rp, _np = _node_projections(node_features, f_sender, f_receiver, g_node)
    ep = _edge_projection(edge_features, f_edge, f_bias)
    edges = jax.nn.relu(ep + sp[senders] + rp[receivers])
    incoming = jnp.maximum(jax.ops.segment_max(edges, receivers, num_segments=n), 0.0)
    outgoing = jnp.maximum(jax.ops.segment_max(edges, senders, num_segments=n), 0.0)
    eagg = jax.ops.segment_sum(edges, edge_graph_ids, num_segments=16)
    nodes, nagg = _final_dense(node_features, incoming, outgoing, node_graph_ids,
                               g_node, g_in, g_out, g_bias)
    globals_ = _globals_dense(eagg, nagg, h_edges, h_nodes, h_bias)
    return nodes, globals_


# speedup vs baseline: 1.6559x; 1.6559x over previous
import functools
import jax
import jax.numpy as jnp
from jax import lax
from jax.experimental import pallas as pl
from jax.experimental.pallas import tpu as pltpu
from jax.experimental.pallas import tpu_sc as plsc

_NT = 32
_E = 320000
_N = 10000
_EB = 128
_NBLK = _E // _EB
_NPT = 313


def _matmul2_kernel(nf_ref, ws_ref, wr_ref, sp_ref, rp_ref):
    x = nf_ref[...]
    sp_ref[...] = jax.lax.dot_general(x, ws_ref[...], (((1,), (1,)), ((), ())),
                                      preferred_element_type=jnp.float32)
    rp_ref[...] = jax.lax.dot_general(x, wr_ref[...], (((1,), (1,)), ((), ())),
                                      preferred_element_type=jnp.float32)


def _node_projections(nf, f_sender, f_receiver):
    n, d = nf.shape
    blk = 1024
    grid = (pl.cdiv(n, blk),)
    out = pl.pallas_call(
        _matmul2_kernel,
        grid=grid,
        in_specs=[
            pl.BlockSpec((blk, d), lambda i: (i, 0)),
            pl.BlockSpec((128, d), lambda i: (0, 0)),
            pl.BlockSpec((128, d), lambda i: (0, 0)),
        ],
        out_specs=[
            pl.BlockSpec((blk, 128), lambda i: (i, 0)),
            pl.BlockSpec((blk, 128), lambda i: (i, 0)),
        ],
        out_shape=[jax.ShapeDtypeStruct((n, 128), jnp.float32)] * 2,
    )(nf, f_sender, f_receiver)
    return out


def _edge_proj_kernel(ef_ref, we_ref, b_ref, out_ref):
    out_ref[...] = jax.lax.dot_general(
        ef_ref[...], we_ref[...], (((1,), (1,)), ((), ())),
        preferred_element_type=jnp.float32) + b_ref[...]


def _edge_projection(ef, f_edge, f_bias):
    e, d = ef.shape
    blk = 2048
    out = pl.pallas_call(
        _edge_proj_kernel,
        grid=(pl.cdiv(e, blk),),
        in_specs=[
            pl.BlockSpec((blk, d), lambda i: (i, 0)),
            pl.BlockSpec((128, d), lambda i: (0, 0)),
            pl.BlockSpec((1, 128), lambda i: (0, 0)),
        ],
        out_specs=pl.BlockSpec((blk, 128), lambda i: (i, 0)),
        out_shape=jax.ShapeDtypeStruct((e, 128), jnp.float32),
    )(ef, f_edge, f_bias.reshape(1, 128))
    return out


def _sread(ref, i):
    return ref[pl.ds(i, 16)][0]


def _sc_edge_body(ep_hbm, sp_hbm, rp_hbm, sidx_hbm, ridx_hbm, bnds_hbm,
                  z_hbm, gacc_hbm, zb, spg, rpg, sidx_v, ridx_v, bnds_v,
                  gacc_v, sem, *, nblocks):
    wid = lax.axis_index("s") * 2 + lax.axis_index("c")
    pltpu.sync_copy(bnds_hbm, bnds_v)
    zeros16 = jnp.zeros((16,), jnp.float32)
    for i in range(128):
        gacc_v[pl.ds(16 * i, 16)] = zeros16
    nblk = nblocks // _NT + jnp.where(wid < nblocks % _NT, 1, 0)

    def blk_body(j, _):
        b = wid + _NT * j
        eb = b * _EB
        pltpu.sync_copy(sidx_hbm.at[pl.ds(eb, _EB)], sidx_v)
        pltpu.sync_copy(ridx_hbm.at[pl.ds(eb, _EB)], ridx_v)
        pltpu.sync_copy(ep_hbm.at[pl.ds(eb, _EB), :], zb)
        pltpu.async_copy(sp_hbm.at[sidx_v], spg, sem).wait()
        pltpu.async_copy(rp_hbm.at[ridx_v], rpg, sem).wait()

        def row(i, c):
            for k in range(8):
                sl = pl.ds(16 * k, 16)
                zb[i, sl] = jnp.maximum(zb[i, sl] + spg[i, sl] + rpg[i, sl], 0.0)
            return c

        lax.fori_loop(0, _EB, row, 0)
        pltpu.sync_copy(zb, z_hbm.at[pl.ds(eb, _EB), :])
        for g in range(16):
            rs = jnp.clip(_sread(bnds_v, g) - eb, 0, _EB)
            re = jnp.clip(_sread(bnds_v, g + 1) - eb, 0, _EB)

            def srow(i, acc):
                return tuple(acc[k] + zb[i, pl.ds(16 * k, 16)] for k in range(8))

            sums = lax.fori_loop(rs, re, srow,
                                 tuple(zeros16 for _ in range(8)))
            for k in range(8):
                sl = pl.ds(g * 128 + 16 * k, 16)
                gacc_v[sl] = gacc_v[sl] + sums[k]
        return _

    lax.fori_loop(0, nblk, blk_body, 0)
    pltpu.sync_copy(gacc_v, gacc_hbm.at[wid])


def _sc_edge_phase(ep, sp, rp, senders, receivers, bnds32):
    e = ep.shape[0]
    nblocks = e // _EB
    mesh = plsc.VectorSubcoreMesh(core_axis_name="c", subcore_axis_name="s")
    body = functools.partial(_sc_edge_body, nblocks=nblocks)
    z, gacc = pl.kernel(
        body,
        out_type=[
            jax.ShapeDtypeStruct((e, 128), jnp.float32),
            jax.ShapeDtypeStruct((_NT, 2048), jnp.float32),
        ],
        mesh=mesh,
        scratch_types=[
            pltpu.VMEM((_EB, 128), jnp.float32),
            pltpu.VMEM((_EB, 128), jnp.float32),
            pltpu.VMEM((_EB, 128), jnp.float32),
            pltpu.VMEM((_EB,), jnp.int32),
            pltpu.VMEM((_EB,), jnp.int32),
            pltpu.VMEM((32,), jnp.int32),
            pltpu.VMEM((2048,), jnp.float32),
            pltpu.SemaphoreType.DMA,
        ],
    )(ep, sp, rp, senders, receivers, bnds32)
    return z, gacc.reshape(_NT, 16, 128).sum(axis=0)


def _final_kernel(nf_ref, inc_ref, out_ref, gids_ref, gn_ref, gi_ref, go_ref,
                  gb_ref, nodes_ref, nagg_ref):
    i = pl.program_id(0)
    dot = functools.partial(jax.lax.dot_general, preferred_element_type=jnp.float32)
    nodes = (dot(nf_ref[...], gn_ref[...], (((1,), (1,)), ((), ())))
             + dot(inc_ref[...], gi_ref[...], (((1,), (1,)), ((), ())))
             + dot(out_ref[...], go_ref[...], (((1,), (1,)), ((), ())))
             + gb_ref[...])
    nodes_ref[...] = nodes
    onehot = (gids_ref[...] == jax.lax.broadcasted_iota(jnp.int32, (16, 1), 0)
              ).astype(jnp.float32)
    part = dot(onehot, jnp.maximum(nodes, 0.0), (((1,), (0,)), ((), ())))

    @pl.when(i == 0)
    def _init():
        nagg_ref[...] = jnp.zeros_like(nagg_ref)

    nagg_ref[...] += part


def _final_dense(nf, incoming, outgoing, node_gids, g_node, g_in, g_out, g_bias):
    n = nf.shape[0]
    blk = 1024
    grid = (pl.cdiv(n, blk),)
    npad = grid[0] * blk
    pad = npad - n
    nf = jnp.pad(nf, ((0, pad), (0, 0)))
    incoming = jnp.pad(incoming, ((0, pad), (0, 0)))
    outgoing = jnp.pad(outgoing, ((0, pad), (0, 0)))
    gids = jnp.full((1, npad), 16, jnp.int32).at[0, :n].set(node_gids.astype(jnp.int32))
    nodes, nagg = pl.pallas_call(
        _final_kernel,
        grid=grid,
        in_specs=[
            pl.BlockSpec((blk, 128), lambda i: (i, 0)),
            pl.BlockSpec((blk, 128), lambda i: (i, 0)),
            pl.BlockSpec((blk, 128), lambda i: (i, 0)),
            pl.BlockSpec((1, blk), lambda i: (0, i)),
            pl.BlockSpec((128, 128), lambda i: (0, 0)),
            pl.BlockSpec((128, 128), lambda i: (0, 0)),
            pl.BlockSpec((128, 128), lambda i: (0, 0)),
            pl.BlockSpec((1, 128), lambda i: (0, 0)),
        ],
        out_specs=[
            pl.BlockSpec((blk, 128), lambda i: (i, 0)),
            pl.BlockSpec((16, 128), lambda i: (0, 0)),
        ],
        out_shape=[
            jax.ShapeDtypeStruct((npad, 128), jnp.float32),
            jax.ShapeDtypeStruct((16, 128), jnp.float32),
        ],
    )(nf, incoming, outgoing, gids, g_node, g_in, g_out, g_bias.reshape(1, 128))
    return nodes[:n], nagg


def _globals_kernel(eagg_ref, nagg_ref, he_ref, hn_ref, hb_ref, out_ref):
    dot = functools.partial(jax.lax.dot_general, preferred_element_type=jnp.float32)
    out_ref[...] = (dot(eagg_ref[...], he_ref[...], (((1,), (1,)), ((), ())))
                    + dot(nagg_ref[...], hn_ref[...], (((1,), (1,)), ((), ())))
                    + hb_ref[...])


def _globals_dense(eagg, nagg, h_edges, h_nodes, h_bias):
    return pl.pallas_call(
        _globals_kernel,
        out_shape=jax.ShapeDtypeStruct((16, 128), jnp.float32),
    )(eagg, nagg, h_edges, h_nodes, h_bias.reshape(1, 128))


def kernel(node_features, edge_features, senders, receivers, node_graph_ids,
           edge_graph_ids, f_edge, f_sender, f_receiver, f_bias, g_node, g_in,
           g_out, g_bias, h_nodes, h_edges, h_bias):
    n = node_features.shape[0]
    sp, rp = _node_projections(node_features, f_sender, f_receiver)
    ep = _edge_projection(edge_features, f_edge, f_bias)
    starts = jnp.searchsorted(edge_graph_ids.astype(jnp.int32),
                              jnp.arange(16, dtype=jnp.int32)).astype(jnp.int32)
    bnds32 = jnp.concatenate([starts, jnp.full((16,), _E, jnp.int32)])
    z, eagg = _sc_edge_phase(ep, sp, rp, senders.astype(jnp.int32),
                             receivers.astype(jnp.int32), bnds32)
    incoming = jnp.maximum(jax.ops.segment_max(z, receivers, num_segments=n), 0.0)
    outgoing = jnp.maximum(jax.ops.segment_max(z, senders, num_segments=n), 0.0)
    nodes, nagg = _final_dense(node_features, incoming, outgoing, node_graph_ids,
                               g_node, g_in, g_out, g_bias)
    globals_ = _globals_dense(eagg, nagg, h_edges, h_nodes, h_bias)
    return nodes, globals_


# speedup vs baseline: 1.9167x; 1.1575x over previous
import functools
import jax
import jax.numpy as jnp
from jax import lax
from jax.experimental import pallas as pl
from jax.experimental.pallas import tpu as pltpu
from jax.experimental.pallas import tpu_sc as plsc

_NT = 32
_E = 320000
_N = 10000
_EB = 128
_NBLK = _E // _EB
_NPT = 320


def _matmul2_kernel(nf_ref, ws_ref, wr_ref, sp_ref, rp_ref):
    x = nf_ref[...]
    sp_ref[...] = jax.lax.dot_general(x, ws_ref[...], (((1,), (1,)), ((), ())),
                                      preferred_element_type=jnp.float32)
    rp_ref[...] = jax.lax.dot_general(x, wr_ref[...], (((1,), (1,)), ((), ())),
                                      preferred_element_type=jnp.float32)


def _node_projections(nf, f_sender, f_receiver):
    n, d = nf.shape
    blk = 1024
    grid = (pl.cdiv(n, blk),)
    out = pl.pallas_call(
        _matmul2_kernel,
        grid=grid,
        in_specs=[
            pl.BlockSpec((blk, d), lambda i: (i, 0)),
            pl.BlockSpec((128, d), lambda i: (0, 0)),
            pl.BlockSpec((128, d), lambda i: (0, 0)),
        ],
        out_specs=[
            pl.BlockSpec((blk, 128), lambda i: (i, 0)),
            pl.BlockSpec((blk, 128), lambda i: (i, 0)),
        ],
        out_shape=[jax.ShapeDtypeStruct((n, 128), jnp.float32)] * 2,
    )(nf, f_sender, f_receiver)
    return out


def _edge_proj_kernel(ef_ref, we_ref, b_ref, out_ref):
    out_ref[...] = jax.lax.dot_general(
        ef_ref[...], we_ref[...], (((1,), (1,)), ((), ())),
        preferred_element_type=jnp.float32) + b_ref[...]


def _edge_projection(ef, f_edge, f_bias):
    e, d = ef.shape
    blk = 2048
    out = pl.pallas_call(
        _edge_proj_kernel,
        grid=(pl.cdiv(e, blk),),
        in_specs=[
            pl.BlockSpec((blk, d), lambda i: (i, 0)),
            pl.BlockSpec((128, d), lambda i: (0, 0)),
            pl.BlockSpec((1, 128), lambda i: (0, 0)),
        ],
        out_specs=pl.BlockSpec((blk, 128), lambda i: (i, 0)),
        out_shape=jax.ShapeDtypeStruct((e, 128), jnp.float32),
    )(ef, f_edge, f_bias.reshape(1, 128))
    return out


def _pack_perm():
    import numpy as _np
    p = _np.zeros(128, _np.int64)
    for k in range(4):
        for j in range(16):
            p[32 * k + 2 * j] = 32 * k + j
            p[32 * k + 2 * j + 1] = 32 * k + 16 + j
    return jnp.asarray(p)


def _sread(ref, i):
    return ref[pl.ds(i, 16)][0]


def _sc_edge_body(ep_hbm, sp_hbm, rp_hbm, sidx_hbm, ridx_hbm, bnds_hbm,
                  z_hbm, gacc_hbm, zb0, zb1, sp0, sp1, rp0, rp1,
                  sidx_a, ridx_a, bnds_v, gacc_v, sem0, sem1, *, nblocks):
    wid = lax.axis_index("s") * 2 + lax.axis_index("c")
    pltpu.sync_copy(bnds_hbm, bnds_v)
    zeros16 = jnp.zeros((16,), jnp.float32)
    for i in range(128):
        gacc_v[pl.ds(16 * i, 16)] = zeros16
    npairs = 39 + jnp.where(wid < 2, 1, 0)
    base_blk = 78 * wid + 2 * jnp.minimum(wid, 2)
    ebase = base_blk * _EB
    pltpu.sync_copy(sidx_hbm.at[pl.ds(ebase, 9984)], sidx_a.at[pl.ds(0, 9984)])
    pltpu.sync_copy(ridx_hbm.at[pl.ds(ebase, 9984)], ridx_a.at[pl.ds(0, 9984)])

    @pl.when(wid < 2)
    def _tail():
        pltpu.sync_copy(sidx_hbm.at[pl.ds(ebase + 9984, 256)],
                        sidx_a.at[pl.ds(9984, 256)])
        pltpu.sync_copy(ridx_hbm.at[pl.ds(ebase + 9984, 256)],
                        ridx_a.at[pl.ds(9984, 256)])

    def fire(j, bz, bs, br, sem):
        eb = ebase + j * _EB
        pltpu.async_copy(ep_hbm.at[pl.ds(eb, _EB), :], bz, sem)
        pltpu.async_copy(sp_hbm.at[sidx_a.at[pl.ds(j * _EB, _EB)]], bs, sem)
        pltpu.async_copy(rp_hbm.at[ridx_a.at[pl.ds(j * _EB, _EB)]], br, sem)

    def drain(j, bz, bs, br, sem):
        eb = ebase + j * _EB
        pltpu.make_async_copy(ep_hbm.at[pl.ds(eb, _EB), :], bz, sem).wait()
        pltpu.make_async_copy(sp_hbm.at[sidx_a.at[pl.ds(j * _EB, _EB)]],
                              bs, sem).wait()
        pltpu.make_async_copy(rp_hbm.at[ridx_a.at[pl.ds(j * _EB, _EB)]],
                              br, sem).wait()

    def compute(j, bz, bs, br):
        eb = ebase + j * _EB

        def row(i, c):
            for k in range(8):
                sl = pl.ds(16 * k, 16)
                bz[i, sl] = jnp.maximum(bz[i, sl] + bs[i, sl] + br[i, sl], 0.0)
            return c

        lax.fori_loop(0, _EB, row, 0)
        pltpu.sync_copy(bz, z_hbm.at[pl.ds(eb, _EB), :])
        for g in range(16):
            rs = jnp.clip(_sread(bnds_v, g) - eb, 0, _EB)
            re = jnp.clip(_sread(bnds_v, g + 1) - eb, 0, _EB)

            def srow(i, acc):
                return tuple(acc[k] + bz[i, pl.ds(16 * k, 16)]
                             for k in range(8))

            sums = lax.fori_loop(rs, re, srow,
                                 tuple(zeros16 for _ in range(8)))
            for k in range(8):
                sl = pl.ds(g * 128 + 16 * k, 16)
                gacc_v[sl] = gacc_v[sl] + sums[k]

    fire(0, zb0, sp0, rp0, sem0)
    fire(1, zb1, sp1, rp1, sem1)

    def pair_body(jj, carry):
        j0 = 2 * jj
        drain(j0, zb0, sp0, rp0, sem0)
        compute(j0, zb0, sp0, rp0)

        @pl.when(jj < npairs - 1)
        def _f0():
            fire(j0 + 2, zb0, sp0, rp0, sem0)

        j1 = j0 + 1
        drain(j1, zb1, sp1, rp1, sem1)
        compute(j1, zb1, sp1, rp1)

        @pl.when(jj < npairs - 1)
        def _f1():
            fire(j1 + 2, zb1, sp1, rp1, sem1)

        return carry

    lax.fori_loop(0, npairs, pair_body, 0)
    pltpu.sync_copy(gacc_v, gacc_hbm.at[wid])


def _sc_edge_phase(ep, sp, rp, senders, receivers, bnds32):
    e = ep.shape[0]
    nblocks = e // _EB
    mesh = plsc.VectorSubcoreMesh(core_axis_name="c", subcore_axis_name="s")
    body = functools.partial(_sc_edge_body, nblocks=nblocks)
    z, gacc = pl.kernel(
        body,
        out_type=[
            jax.ShapeDtypeStruct((e, 128), jnp.float32),
            jax.ShapeDtypeStruct((_NT, 2048), jnp.float32),
        ],
        mesh=mesh,
        scratch_types=[
            pltpu.VMEM((_EB, 128), jnp.float32),
            pltpu.VMEM((_EB, 128), jnp.float32),
            pltpu.VMEM((_EB, 128), jnp.float32),
            pltpu.VMEM((_EB, 128), jnp.float32),
            pltpu.VMEM((_EB, 128), jnp.float32),
            pltpu.VMEM((_EB, 128), jnp.float32),
            pltpu.VMEM((10240,), jnp.int32),
            pltpu.VMEM((10240,), jnp.int32),
            pltpu.VMEM((32,), jnp.int32),
            pltpu.VMEM((2048,), jnp.float32),
            pltpu.SemaphoreType.DMA,
            pltpu.SemaphoreType.DMA,
        ],
    )(ep, sp, rp, senders, receivers, bnds32)
    return z, gacc.reshape(_NT, 16, 128).sum(axis=0)


def _sc_max_body(z_hbm, sidx_hbm, ridx_hbm, min_hbm, mout_hbm,
                 minv, moutv, schunk, rchunk, eid_in, rl_in, eid_out, rl_out,
                 zbatch, tmp16, sem, *, nchunks, chunk):
    wid = lax.axis_index("s") * 2 + lax.axis_index("c")
    lo = wid * _NPT
    zeros16 = jnp.zeros((16,), jnp.float32)
    izeros = jnp.zeros((16,), jnp.int32)

    def zrow(i, c):
        for k in range(8):
            minv[i, pl.ds(16 * k, 16)] = zeros16
            moutv[i, pl.ds(16 * k, 16)] = zeros16
        return c

    lax.fori_loop(0, _NPT, zrow, 0)
    cap = chunk + 16

    def zlist(i, c):
        sl = pl.ds(16 * i, 16)
        eid_in[sl] = izeros
        eid_out[sl] = izeros
        return c

    lax.fori_loop(0, cap // 16, zlist, 0)

    lo16 = jnp.full((16,), lo, jnp.int32)
    hi16 = jnp.full((16,), lo + _NPT, jnp.int32)
    iota16 = lax.iota(jnp.int32, 16)

    def chunk_body(c, carry):
        base = c * chunk
        pltpu.sync_copy(ridx_hbm.at[pl.ds(base, chunk)], rchunk)
        pltpu.sync_copy(sidx_hbm.at[pl.ds(base, chunk)], schunk)

        def grp(g, cnts):
            cinv, coutv = cnts
            ev = jnp.full((16,), base + 16 * g, jnp.int32) + iota16
            r = rchunk[pl.ds(16 * g, 16)]
            m = (r >= lo16) & (r < hi16)
            pos = plsc.cumsum(m.astype(jnp.int32)) - 1 + cinv
            plsc.store_scatter(eid_in, [pos], ev, mask=m)
            plsc.store_scatter(rl_in, [pos], r - lo16, mask=m)
            cinv = cinv + plsc.all_reduce_population_count(m)
            s = schunk[pl.ds(16 * g, 16)]
            m2 = (s >= lo16) & (s < hi16)
            pos2 = plsc.cumsum(m2.astype(jnp.int32)) - 1 + coutv
            plsc.store_scatter(eid_out, [pos2], ev, mask=m2)
            plsc.store_scatter(rl_out, [pos2], s - lo16, mask=m2)
            coutv = coutv + plsc.all_reduce_population_count(m2)
            return (cinv, coutv)

        zero16i = jnp.zeros((16,), jnp.int32)
        cinv, coutv = lax.fori_loop(0, chunk // 16, grp, (zero16i, zero16i))
        tmp16[pl.ds(0, 16)] = cinv
        tmp16[pl.ds(16, 16)] = coutv
        cin = _sread(tmp16, 0)
        cout = _sread(tmp16, 16)

        def make_batch(eidref, rlref, cnt, accref):
            def batch(b, c2):
                pltpu.async_copy(z_hbm.at[eidref.at[pl.ds(128 * b, 128)]],
                                 zbatch, sem).wait()
                nupd = jnp.minimum(cnt - 128 * b, 128)

                def upd(j, c3):
                    rl = _sread(rlref, 128 * b + j)
                    for k in range(8):
                        sl = pl.ds(16 * k, 16)
                        accref[rl, sl] = jnp.maximum(accref[rl, sl],
                                                     zbatch[j, sl])
                    return c3

                lax.fori_loop(0, nupd, upd, 0)
                return c2

            return batch

        lax.fori_loop(0, (cin + 127) // 128,
                      make_batch(eid_in, rl_in, cin, minv), 0)
        lax.fori_loop(0, (cout + 127) // 128,
                      make_batch(eid_out, rl_out, cout, moutv), 0)
        return carry

    lax.fori_loop(0, nchunks, chunk_body, 0)
    pltpu.sync_copy(minv, min_hbm.at[pl.ds(lo, _NPT), :])
    pltpu.sync_copy(moutv, mout_hbm.at[pl.ds(lo, _NPT), :])


def _sc_segment_max(z, senders, receivers):
    chunk = 4000
    nchunks = z.shape[0] // chunk
    cap = chunk + 16
    mesh = plsc.VectorSubcoreMesh(core_axis_name="c", subcore_axis_name="s")
    body = functools.partial(_sc_max_body, nchunks=nchunks, chunk=chunk)
    mn, mo = pl.kernel(
        body,
        out_type=[
            jax.ShapeDtypeStruct((_NT * _NPT, 128), jnp.float32),
            jax.ShapeDtypeStruct((_NT * _NPT, 128), jnp.float32),
        ],
        mesh=mesh,
        scratch_types=[
            pltpu.VMEM((_NPT, 128), jnp.float32),
            pltpu.VMEM((_NPT, 128), jnp.float32),
            pltpu.VMEM((chunk,), jnp.int32),
            pltpu.VMEM((chunk,), jnp.int32),
            pltpu.VMEM((cap,), jnp.int32),
            pltpu.VMEM((cap,), jnp.int32),
            pltpu.VMEM((cap,), jnp.int32),
            pltpu.VMEM((cap,), jnp.int32),
            pltpu.VMEM((128, 128), jnp.float32),
            pltpu.VMEM((32,), jnp.int32),
            pltpu.SemaphoreType.DMA,
        ],
    )(z, senders, receivers)
    return mn, mo


def _final_kernel(nf_ref, inc_ref, out_ref, gids_ref, gn_ref, gi_ref, go_ref,
                  gb_ref, nodes_ref, nagg_ref):
    i = pl.program_id(0)
    dot = functools.partial(jax.lax.dot_general, preferred_element_type=jnp.float32)
    nodes = (dot(nf_ref[...], gn_ref[...], (((1,), (1,)), ((), ())))
             + dot(inc_ref[...], gi_ref[...], (((1,), (1,)), ((), ())))
             + dot(out_ref[...], go_ref[...], (((1,), (1,)), ((), ())))
             + gb_ref[...])
    nodes_ref[...] = nodes
    onehot = (gids_ref[...] == jax.lax.broadcasted_iota(jnp.int32, (16, 1), 0)
              ).astype(jnp.float32)
    part = dot(onehot, jnp.maximum(nodes, 0.0), (((1,), (0,)), ((), ())))

    @pl.when(i == 0)
    def _init():
        nagg_ref[...] = jnp.zeros_like(nagg_ref)

    nagg_ref[...] += part


def _final_dense(nf, incoming, outgoing, node_gids, g_node, g_in, g_out, g_bias):
    n = nf.shape[0]
    blk = 1024
    grid = (pl.cdiv(n, blk),)
    npad = grid[0] * blk
    pad = npad - n
    nf = jnp.pad(nf, ((0, pad), (0, 0)))
    incoming = jnp.pad(incoming, ((0, pad), (0, 0)))
    outgoing = jnp.pad(outgoing, ((0, pad), (0, 0)))
    gids = jnp.full((1, npad), 16, jnp.int32).at[0, :n].set(node_gids.astype(jnp.int32))
    nodes, nagg = pl.pallas_call(
        _final_kernel,
        grid=grid,
        in_specs=[
            pl.BlockSpec((blk, 128), lambda i: (i, 0)),
            pl.BlockSpec((blk, 128), lambda i: (i, 0)),
            pl.BlockSpec((blk, 128), lambda i: (i, 0)),
            pl.BlockSpec((1, blk), lambda i: (0, i)),
            pl.BlockSpec((128, 128), lambda i: (0, 0)),
            pl.BlockSpec((128, 128), lambda i: (0, 0)),
            pl.BlockSpec((128, 128), lambda i: (0, 0)),
            pl.BlockSpec((1, 128), lambda i: (0, 0)),
        ],
        out_specs=[
            pl.BlockSpec((blk, 128), lambda i: (i, 0)),
            pl.BlockSpec((16, 128), lambda i: (0, 0)),
        ],
        out_shape=[
            jax.ShapeDtypeStruct((npad, 128), jnp.float32),
            jax.ShapeDtypeStruct((16, 128), jnp.float32),
        ],
    )(nf, incoming, outgoing, gids, g_node, g_in, g_out, g_bias.reshape(1, 128))
    return nodes[:n], nagg


def _globals_kernel(eagg_ref, nagg_ref, he_ref, hn_ref, hb_ref, out_ref):
    dot = functools.partial(jax.lax.dot_general, preferred_element_type=jnp.float32)
    out_ref[...] = (dot(eagg_ref[...], he_ref[...], (((1,), (1,)), ((), ())))
                    + dot(nagg_ref[...], hn_ref[...], (((1,), (1,)), ((), ())))
                    + hb_ref[...])


def _globals_dense(eagg, nagg, h_edges, h_nodes, h_bias):
    return pl.pallas_call(
        _globals_kernel,
        out_shape=jax.ShapeDtypeStruct((16, 128), jnp.float32),
    )(eagg, nagg, h_edges, h_nodes, h_bias.reshape(1, 128))


def kernel(node_features, edge_features, senders, receivers, node_graph_ids,
           edge_graph_ids, f_edge, f_sender, f_receiver, f_bias, g_node, g_in,
           g_out, g_bias, h_nodes, h_edges, h_bias):
    n = node_features.shape[0]
    sp, rp = _node_projections(node_features, f_sender, f_receiver)
    ep = _edge_projection(edge_features, f_edge, f_bias)
    starts = jnp.searchsorted(edge_graph_ids.astype(jnp.int32),
                              jnp.arange(16, dtype=jnp.int32)).astype(jnp.int32)
    bnds32 = jnp.concatenate([starts, jnp.full((16,), _E, jnp.int32)])
    z, eagg = _sc_edge_phase(ep, sp, rp, senders.astype(jnp.int32),
                             receivers.astype(jnp.int32), bnds32)
    incoming = jnp.maximum(jax.ops.segment_max(z, receivers, num_segments=n), 0.0)
    outgoing = jnp.maximum(jax.ops.segment_max(z, senders, num_segments=n), 0.0)
    nodes, nagg = _final_dense(node_features, incoming, outgoing, node_graph_ids,
                               g_node, g_in, g_out, g_bias)
    globals_ = _globals_dense(eagg, nagg, h_edges, h_nodes, h_bias)
    return nodes, globals_
